# Initial kernel scaffold; baseline (speedup 1.0000x reference)
import jax
import jax.numpy as jnp
from jax.experimental import pallas as pl


def kernel(node_features, edge_features, inverse_edge_features, edge_index, inverse_edge_index, params):
    raise NotImplementedError("write your pallas kernel here")


# speedup vs baseline: 2.8212x; 2.8212x over previous
import jax
import jax.numpy as jnp
from jax import lax
from jax.experimental import pallas as pl
from jax.experimental.pallas import tpu as pltpu
from jax.experimental.pallas import tpu_sc as plsc

N = 10000
E = 320000
D = 128
NL = 6
NSUB = 16
NCORE = 2
CHUNK = 128
SUP = 8
NSUP = 20
KC_G = NSUP * SUP
EPAD_G = NSUB * KC_G * CHUNK
ET = E // NSUB
KC_S = ET // CHUNK
TAIL = ET - KC_S * CHUNK
KC_SP = KC_S + 1
NPAD = 10112
DUMMY = N
RZ = NPAD // NSUB


def _sc_gather_body(x_hbm, srci_hbm, dsti_hbm, zeros_hbm, g_hbm,
                    src_v, dst_v, rows0, rows1, acc_sh, sem0, sem1):
    cid = lax.axis_index("c")
    sid = lax.axis_index("s")
    pltpu.sync_copy(zeros_hbm.at[pl.ds(sid * RZ, RZ)],
                    acc_sh.at[pl.ds(sid * RZ, RZ)])
    plsc.subcore_barrier()

    def body(o, carry):
        pltpu.sync_copy(srci_hbm.at[cid, sid, pl.ds(o * SUP, SUP)], src_v)
        pltpu.sync_copy(dsti_hbm.at[cid, sid, pl.ds(o * SUP, SUP)], dst_v)
        for m in range(SUP // 2):
            j0 = 2 * m
            j1 = j0 + 1
            cp0 = pltpu.async_copy(x_hbm.at[src_v.at[j0]], rows0, sem0)
            cp1 = pltpu.async_copy(x_hbm.at[src_v.at[j1]], rows1, sem1)
            cp0.wait()
            pltpu.sync_copy(rows0, acc_sh.at[dst_v.at[j0]], add=True)
            cp1.wait()
            pltpu.sync_copy(rows1, acc_sh.at[dst_v.at[j1]], add=True)
        return carry

    lax.fori_loop(0, NSUP, body, 0)
    plsc.subcore_barrier()
    pltpu.sync_copy(acc_sh.at[pl.ds(sid * RZ, RZ)],
                    g_hbm.at[cid, pl.ds(sid * RZ, RZ)])


_sc_gather = pl.kernel(
    _sc_gather_body,
    out_type=jax.ShapeDtypeStruct((NCORE, NPAD, D), jnp.float32),
    mesh=plsc.VectorSubcoreMesh(core_axis_name="c", subcore_axis_name="s"),
    scratch_types=[
        pltpu.VMEM((SUP, CHUNK), jnp.int32),
        pltpu.VMEM((SUP, CHUNK), jnp.int32),
        pltpu.VMEM((CHUNK, D), jnp.float32),
        pltpu.VMEM((CHUNK, D), jnp.float32),
        pltpu.VMEM_SHARED((NPAD, D), jnp.float32),
        pltpu.SemaphoreType.DMA,
        pltpu.SemaphoreType.DMA,
    ],
)


def _sc_segsum_body(r_hbm, ir_hbm, dsti_hbm, zeros_hbm, s_hbm,
                    dst_v, rows0, acc_sh, sem0):
    cid = lax.axis_index("c")
    sid = lax.axis_index("s")
    pltpu.sync_copy(zeros_hbm.at[pl.ds(sid * RZ, RZ)],
                    acc_sh.at[pl.ds(sid * RZ, RZ)])
    pltpu.sync_copy(dsti_hbm.at[cid, sid], dst_v)
    plsc.subcore_barrier()
    base = sid * ET

    def run(src_ref):
        def body(j, carry):
            pltpu.sync_copy(src_ref.at[pl.ds(base + j * CHUNK, CHUNK)], rows0)
            pltpu.sync_copy(rows0, acc_sh.at[dst_v.at[j]], add=True)
            return carry

        lax.fori_loop(0, KC_S, body, 0)
        pltpu.sync_copy(zeros_hbm.at[pl.ds(0, CHUNK - TAIL)],
                        rows0.at[pl.ds(TAIL, CHUNK - TAIL)])
        pltpu.sync_copy(src_ref.at[pl.ds(base + KC_S * CHUNK, TAIL)],
                        rows0.at[pl.ds(0, TAIL)])
        pltpu.sync_copy(rows0, acc_sh.at[dst_v.at[KC_S]], add=True)

    @pl.when(cid == 0)
    def _():
        run(r_hbm)

    @pl.when(cid == 1)
    def _():
        run(ir_hbm)

    plsc.subcore_barrier()
    pltpu.sync_copy(acc_sh.at[pl.ds(sid * RZ, RZ)],
                    s_hbm.at[cid, pl.ds(sid * RZ, RZ)])


_sc_segsum = pl.kernel(
    _sc_segsum_body,
    out_type=jax.ShapeDtypeStruct((NCORE, NPAD, D), jnp.float32),
    mesh=plsc.VectorSubcoreMesh(core_axis_name="c", subcore_axis_name="s"),
    scratch_types=[
        pltpu.VMEM((KC_SP, CHUNK), jnp.int32),
        pltpu.VMEM((CHUNK, D), jnp.float32),
        pltpu.VMEM_SHARED((NPAD, D), jnp.float32),
        pltpu.SemaphoreType.DMA,
    ],
)


def _wprep_body(wrel_ref, win_ref, wout_ref, ab_ref, c6_ref):
    ii = lax.broadcasted_iota(jnp.int32, (D, D), 0)
    jj = lax.broadcasted_iota(jnp.int32, (D, D), 1)
    c = (ii == jj).astype(jnp.float32)
    for l in range(NL):
        ab_ref[0, l] = jnp.dot(c, win_ref[l], preferred_element_type=jnp.float32)
        ab_ref[1, l] = jnp.dot(c, wout_ref[l], preferred_element_type=jnp.float32)
        c = jnp.dot(c, wrel_ref[l], preferred_element_type=jnp.float32)
    c6_ref[...] = c


_wprep = pl.pallas_call(
    _wprep_body,
    out_shape=(jax.ShapeDtypeStruct((2, NL, D, D), jnp.float32),
               jax.ShapeDtypeStruct((D, D), jnp.float32)),
)


RB = 2000


def _dense_body(g_ref, s_ref, x_ref, w_ref, b_ref, o_ref):
    acc = jnp.dot(g_ref[0], w_ref[0], preferred_element_type=jnp.float32)
    acc += jnp.dot(g_ref[1], w_ref[1], preferred_element_type=jnp.float32)
    acc += jnp.dot(x_ref[...], w_ref[2], preferred_element_type=jnp.float32)
    acc -= jnp.dot(s_ref[0], w_ref[3], preferred_element_type=jnp.float32)
    acc -= jnp.dot(s_ref[1], w_ref[4], preferred_element_type=jnp.float32)
    o_ref[...] = jnp.tanh(acc / 3.0 + b_ref[...])


_dense = pl.pallas_call(
    _dense_body,
    grid=(N // RB,),
    in_specs=[
        pl.BlockSpec((2, RB, D), lambda i: (0, i, 0)),
        pl.BlockSpec((2, RB, D), lambda i: (0, i, 0)),
        pl.BlockSpec((RB, D), lambda i: (i, 0)),
        pl.BlockSpec((5, D, D), lambda i: (0, 0, 0)),
        pl.BlockSpec((1, D), lambda i: (0, 0)),
    ],
    out_specs=pl.BlockSpec((RB, D), lambda i: (i, 0)),
    out_shape=jax.ShapeDtypeStruct((N, D), jnp.float32),
)

EB = 3200


def _relmm_body(r_ref, c_ref, o_ref):
    o_ref[...] = jnp.dot(r_ref[...], c_ref[...],
                         preferred_element_type=jnp.float32)


_relmm = pl.pallas_call(
    _relmm_body,
    grid=(E // EB,),
    in_specs=[pl.BlockSpec((EB, D), lambda i: (i, 0)),
              pl.BlockSpec((D, D), lambda i: (0, 0))],
    out_specs=pl.BlockSpec((EB, D), lambda i: (i, 0)),
    out_shape=jax.ShapeDtypeStruct((E, D), jnp.float32),
)


def kernel(node_features, edge_features, inverse_edge_features,
           edge_index, inverse_edge_index, params):
    x = node_features
    r0 = edge_features
    ir0 = inverse_edge_features
    ei = edge_index.astype(jnp.int32)
    iei = inverse_edge_index.astype(jnp.int32)
    src, dst = ei[0], ei[1]
    isrc, idst = iei[0], iei[1]

    def pack_g(a, padval):
        return jnp.pad(a, (0, EPAD_G - E),
                       constant_values=padval).reshape(NSUB, KC_G, CHUNK)

    def pack_s(a):
        return jnp.pad(a.reshape(NSUB, ET), ((0, 0), (0, CHUNK - TAIL)),
                       constant_values=DUMMY).reshape(NSUB, KC_SP, CHUNK)

    srcp = jnp.stack([pack_g(src, 0), pack_g(isrc, 0)])
    dstp = jnp.stack([pack_g(dst, DUMMY), pack_g(idst, DUMMY)])
    dsts = jnp.stack([pack_s(dst), pack_s(idst)])
    zeros = jnp.zeros((NPAD, D), jnp.float32)

    wrel = jnp.stack([p['W_rel'] for p in params])
    win = jnp.stack([p['W_in'] for p in params])
    wout = jnp.stack([p['W_out'] for p in params])
    ab, c6 = _wprep(wrel, win, wout)

    s = _sc_segsum(r0, ir0, dsts, zeros)
    x, s = lax.optimization_barrier((x, s))
    for l in range(NL):
        g = _sc_gather(x, srcp, dstp, zeros)
        w5 = jnp.stack([params[l]['W_in'], params[l]['W_out'],
                        params[l]['W_loop'], ab[0, l], ab[1, l]])
        x = _dense(g, s, x, w5, params[l]['b'].reshape(1, D))

    r_out = _relmm(r0, c6)
    ir_out = _relmm(ir0, c6)
    return x, r_out, ir_out
